# baseline (speedup 1.0000x reference)
import functools

import jax
import jax.numpy as jnp
from jax.experimental import pallas as pl
from jax.experimental.pallas import tpu as pltpu

_INT_MIN = -2147483648


def _dense_pass_kernel(conf_ref, lab_ref, pred_ref, gt_ref,
                       mining_ref, ce_ref, sl1_ref):
    step = pl.program_id(0)
    x = conf_ref[...]
    tp, c = x.shape
    lab = lab_ref[...]
    m = jnp.max(x, axis=1, keepdims=True)
    s = jnp.sum(jnp.exp(x - m), axis=1, keepdims=True)
    lse = m + jnp.log(s)
    lane = jax.lax.broadcasted_iota(jnp.int32, (tp, c), 1)
    x0 = x[:, 0:1]
    xl = jnp.sum(jnp.where(lane == lab, x, 0.0), axis=1, keepdims=True)
    mining_ref[...] = lse - x0
    ce_ref[...] = lse - xl

    d = pred_ref[...] - gt_ref[...]
    ad = jnp.abs(d)
    sl1 = jnp.where(ad < 1.0, 0.5 * d * d, ad - 0.5)
    row_sl1 = jnp.sum(sl1, axis=1, keepdims=True)
    part = jnp.sum(jnp.where(lab > 0, row_sl1, 0.0))

    @pl.when(step == 0)
    def _():
        sl1_ref[...] = jnp.zeros((1, 1), jnp.float32)

    sl1_ref[...] += part.reshape(1, 1)


def _sum11(v):
    return jnp.sum(jnp.sum(v, axis=1, keepdims=True), axis=0, keepdims=True)


def _select_pass_kernel(mining_ref, ce_ref, lab_ref, sl1_ref,
                        reg_ref, cls_ref, tot_ref, acc_cls, acc_pos, *,
                        batch, neg_pos_ratio):
    i = pl.program_id(0)
    mining = mining_ref[0]
    ce = ce_ref[0]
    lab = lab_ref[0]
    rows, l = mining.shape
    p = rows * l

    pos = lab > 0
    num_pos = _sum11(pos.astype(jnp.int32))
    quota = num_pos * neg_pos_ratio
    neg_count = p - num_pos
    take_all = quota >= neg_count

    ib = jax.lax.bitcast_convert_type(mining, jnp.int32)
    key = jnp.where(pos, jnp.full_like(ib, _INT_MIN), ib)

    def vbody(j, t):
        cand = t | jax.lax.shift_left(jnp.int32(1), jnp.int32(30) - j)
        cnt = _sum11((key >= cand).astype(jnp.int32))
        return jnp.where(cnt >= quota, cand, t)

    t = jax.lax.fori_loop(0, 31, vbody, jnp.zeros((1, 1), jnp.int32))
    count_gt = _sum11((key > t).astype(jnp.int32))
    r = quota - count_gt
    tie = key == t
    idx = (jax.lax.broadcasted_iota(jnp.int32, (rows, l), 0) * l
           + jax.lax.broadcasted_iota(jnp.int32, (rows, l), 1))

    def ibody(j, s):
        cand = s | jax.lax.shift_left(jnp.int32(1), jnp.int32(13) - j)
        cnt = _sum11((tie & (idx < cand)).astype(jnp.int32))
        return jnp.where(cnt < r, cand, s)

    s = jax.lax.fori_loop(0, 14, ibody, jnp.zeros((1, 1), jnp.int32))

    inc = pos | take_all | (key > t) | (tie & (idx <= s) & (r > 0))
    part_cls = _sum11(jnp.where(inc, ce, 0.0))

    @pl.when(i == 0)
    def _():
        acc_cls[...] = jnp.zeros((1, 1), jnp.float32)
        acc_pos[...] = jnp.zeros((1, 1), jnp.float32)

    acc_cls[...] += part_cls
    acc_pos[...] += num_pos.astype(jnp.float32)

    @pl.when(i == batch - 1)
    def _():
        denom = jnp.maximum(acc_pos[...], 1.0)
        reg = sl1_ref[...] / denom
        cls = acc_cls[...] / denom
        reg_ref[...] = reg
        cls_ref[...] = cls
        tot_ref[...] = reg + cls


def kernel(confidence, predicted_locations, gt_locations, labels):
    b, p, c = confidence.shape
    n = b * p
    neg_pos_ratio = 3

    conf2 = confidence.reshape(n, c)
    pred2 = predicted_locations.reshape(n, 4)
    gt2 = gt_locations.reshape(n, 4)
    lab_i32 = labels.astype(jnp.int32)
    lab_col = lab_i32.reshape(n, 1)

    tp = 2048
    grid = (n // tp,)
    mining, ce, sl1 = pl.pallas_call(
        _dense_pass_kernel,
        grid=grid,
        in_specs=[
            pl.BlockSpec((tp, c), lambda i: (i, 0)),
            pl.BlockSpec((tp, 1), lambda i: (i, 0)),
            pl.BlockSpec((tp, 4), lambda i: (i, 0)),
            pl.BlockSpec((tp, 4), lambda i: (i, 0)),
        ],
        out_specs=[
            pl.BlockSpec((tp, 1), lambda i: (i, 0)),
            pl.BlockSpec((tp, 1), lambda i: (i, 0)),
            pl.BlockSpec((1, 1), lambda i: (0, 0)),
        ],
        out_shape=[
            jax.ShapeDtypeStruct((n, 1), jnp.float32),
            jax.ShapeDtypeStruct((n, 1), jnp.float32),
            jax.ShapeDtypeStruct((1, 1), jnp.float32),
        ],
    )(conf2, lab_col, pred2, gt2)

    rows = p // 128
    mining3 = mining.reshape(b, rows, 128)
    ce3 = ce.reshape(b, rows, 128)
    lab3 = lab_i32.reshape(b, rows, 128)

    reg, cls, tot = pl.pallas_call(
        functools.partial(_select_pass_kernel, batch=b,
                          neg_pos_ratio=neg_pos_ratio),
        grid=(b,),
        in_specs=[
            pl.BlockSpec((1, rows, 128), lambda i: (i, 0, 0)),
            pl.BlockSpec((1, rows, 128), lambda i: (i, 0, 0)),
            pl.BlockSpec((1, rows, 128), lambda i: (i, 0, 0)),
            pl.BlockSpec((1, 1), lambda i: (0, 0)),
        ],
        out_specs=[
            pl.BlockSpec((1, 1), lambda i: (0, 0)),
            pl.BlockSpec((1, 1), lambda i: (0, 0)),
            pl.BlockSpec((1, 1), lambda i: (0, 0)),
        ],
        out_shape=[
            jax.ShapeDtypeStruct((1, 1), jnp.float32),
            jax.ShapeDtypeStruct((1, 1), jnp.float32),
            jax.ShapeDtypeStruct((1, 1), jnp.float32),
        ],
        scratch_shapes=[
            pltpu.VMEM((1, 1), jnp.float32),
            pltpu.VMEM((1, 1), jnp.float32),
        ],
    )(mining3, ce3, lab3, sl1)

    return (reg[0, 0], cls[0, 0], tot[0, 0])


# speedup vs baseline: 1.1721x; 1.1721x over previous
import functools

import jax
import jax.numpy as jnp
from jax.experimental import pallas as pl
from jax.experimental.pallas import tpu as pltpu

_INT_MIN = -2147483648


def _dense_pass_kernel(conf_ref, lab_ref, pred_ref, gt_ref,
                       mining_ref, ce_ref, sl1_ref):
    step = pl.program_id(0)
    x = conf_ref[...]
    bq, l, c = x.shape
    lab = lab_ref[...]
    m = jnp.max(x, axis=2)
    s = jnp.sum(jnp.exp(x - m[:, :, None]), axis=2)
    lse = m + jnp.log(s)
    lane = jax.lax.broadcasted_iota(jnp.int32, (bq, l, c), 2)
    x0 = x[:, :, 0]
    xl = jnp.sum(jnp.where(lane == lab[:, :, None], x, 0.0), axis=2)
    mining_ref[...] = lse - x0
    ce_ref[...] = lse - xl

    d = pred_ref[...] - gt_ref[...]
    ad = jnp.abs(d)
    sl1 = jnp.where(ad < 1.0, 0.5 * d * d, ad - 0.5)
    row_sl1 = jnp.sum(sl1, axis=2)
    part = jnp.sum(jnp.where(lab > 0, row_sl1, 0.0))

    @pl.when(step == 0)
    def _():
        sl1_ref[...] = jnp.zeros((1, 1), jnp.float32)

    sl1_ref[...] += part.reshape(1, 1)


def _sum11(v):
    return jnp.sum(jnp.sum(v, axis=1, keepdims=True), axis=0, keepdims=True)


def _select_pass_kernel(mining_ref, ce_ref, lab_ref, sl1_ref,
                        reg_ref, cls_ref, tot_ref, acc_cls, acc_pos, *,
                        batch, neg_pos_ratio):
    i = pl.program_id(0)
    mining = mining_ref[0]
    ce = ce_ref[0]
    lab = lab_ref[0]
    rows, l = mining.shape
    p = rows * l

    pos = lab > 0
    num_pos = _sum11(pos.astype(jnp.int32))
    quota = num_pos * neg_pos_ratio
    neg_count = p - num_pos
    take_all = quota >= neg_count

    ib = jax.lax.bitcast_convert_type(mining, jnp.int32)
    key = jnp.where(pos, jnp.full_like(ib, _INT_MIN), ib)

    def vbody(j, t):
        cand = t | jax.lax.shift_left(jnp.int32(1), jnp.int32(30) - j)
        cnt = _sum11((key >= cand).astype(jnp.int32))
        return jnp.where(cnt >= quota, cand, t)

    t = jax.lax.fori_loop(0, 31, vbody, jnp.zeros((1, 1), jnp.int32))
    count_gt = _sum11((key > t).astype(jnp.int32))
    r = quota - count_gt
    tie = key == t
    idx = (jax.lax.broadcasted_iota(jnp.int32, (rows, l), 0) * l
           + jax.lax.broadcasted_iota(jnp.int32, (rows, l), 1))

    def ibody(j, s):
        cand = s | jax.lax.shift_left(jnp.int32(1), jnp.int32(13) - j)
        cnt = _sum11((tie & (idx < cand)).astype(jnp.int32))
        return jnp.where(cnt < r, cand, s)

    s = jax.lax.fori_loop(0, 14, ibody, jnp.zeros((1, 1), jnp.int32))

    inc = pos | take_all | (key > t) | (tie & (idx <= s) & (r > 0))
    part_cls = _sum11(jnp.where(inc, ce, 0.0))

    @pl.when(i == 0)
    def _():
        acc_cls[...] = jnp.zeros((1, 1), jnp.float32)
        acc_pos[...] = jnp.zeros((1, 1), jnp.float32)

    acc_cls[...] += part_cls
    acc_pos[...] += num_pos.astype(jnp.float32)

    @pl.when(i == batch - 1)
    def _():
        denom = jnp.maximum(acc_pos[...], 1.0)
        reg = sl1_ref[...] / denom
        cls = acc_cls[...] / denom
        reg_ref[...] = reg
        cls_ref[...] = cls
        tot_ref[...] = reg + cls


def kernel(confidence, predicted_locations, gt_locations, labels):
    b, p, c = confidence.shape
    n = b * p
    q = n // 128
    neg_pos_ratio = 3

    conf3 = confidence.reshape(q, 128, c)
    pred3 = predicted_locations.reshape(q, 128, 4)
    gt3 = gt_locations.reshape(q, 128, 4)
    lab_i32 = labels.astype(jnp.int32)
    lab2 = lab_i32.reshape(q, 128)

    bq = 16
    grid = (q // bq,)
    mining, ce, sl1 = pl.pallas_call(
        _dense_pass_kernel,
        grid=grid,
        in_specs=[
            pl.BlockSpec((bq, 128, c), lambda i: (i, 0, 0)),
            pl.BlockSpec((bq, 128), lambda i: (i, 0)),
            pl.BlockSpec((bq, 128, 4), lambda i: (i, 0, 0)),
            pl.BlockSpec((bq, 128, 4), lambda i: (i, 0, 0)),
        ],
        out_specs=[
            pl.BlockSpec((bq, 128), lambda i: (i, 0)),
            pl.BlockSpec((bq, 128), lambda i: (i, 0)),
            pl.BlockSpec((1, 1), lambda i: (0, 0)),
        ],
        out_shape=[
            jax.ShapeDtypeStruct((q, 128), jnp.float32),
            jax.ShapeDtypeStruct((q, 128), jnp.float32),
            jax.ShapeDtypeStruct((1, 1), jnp.float32),
        ],
    )(conf3, lab2, pred3, gt3)

    rows = p // 128
    mining3 = mining.reshape(b, rows, 128)
    ce3 = ce.reshape(b, rows, 128)
    lab3 = lab_i32.reshape(b, rows, 128)

    reg, cls, tot = pl.pallas_call(
        functools.partial(_select_pass_kernel, batch=b,
                          neg_pos_ratio=neg_pos_ratio),
        grid=(b,),
        in_specs=[
            pl.BlockSpec((1, rows, 128), lambda i: (i, 0, 0)),
            pl.BlockSpec((1, rows, 128), lambda i: (i, 0, 0)),
            pl.BlockSpec((1, rows, 128), lambda i: (i, 0, 0)),
            pl.BlockSpec((1, 1), lambda i: (0, 0)),
        ],
        out_specs=[
            pl.BlockSpec((1, 1), lambda i: (0, 0)),
            pl.BlockSpec((1, 1), lambda i: (0, 0)),
            pl.BlockSpec((1, 1), lambda i: (0, 0)),
        ],
        out_shape=[
            jax.ShapeDtypeStruct((1, 1), jnp.float32),
            jax.ShapeDtypeStruct((1, 1), jnp.float32),
            jax.ShapeDtypeStruct((1, 1), jnp.float32),
        ],
        scratch_shapes=[
            pltpu.VMEM((1, 1), jnp.float32),
            pltpu.VMEM((1, 1), jnp.float32),
        ],
    )(mining3, ce3, lab3, sl1)

    return (reg[0, 0], cls[0, 0], tot[0, 0])


# speedup vs baseline: 1.2047x; 1.0278x over previous
import functools

import jax
import jax.numpy as jnp
from jax.experimental import pallas as pl
from jax.experimental.pallas import tpu as pltpu

_INT_MIN = -2147483648


def _dense_pass_kernel(conf_ref, lab_ref, pred_ref, gt_ref,
                       mining_ref, ce_ref, sl1_ref):
    step = pl.program_id(0)
    x = conf_ref[...]
    bq, l, c = x.shape
    lab = lab_ref[...]
    m = jnp.max(x, axis=2)
    s = jnp.sum(jnp.exp(x - m[:, :, None]), axis=2)
    lse = m + jnp.log(s)
    lane = jax.lax.broadcasted_iota(jnp.int32, (bq, l, c), 2)
    x0 = x[:, :, 0]
    xl = jnp.sum(jnp.where(lane == lab[:, :, None], x, 0.0), axis=2)
    mining_ref[...] = lse - x0
    ce_ref[...] = lse - xl

    d = pred_ref[...] - gt_ref[...]
    ad = jnp.abs(d)
    sl1 = jnp.where(ad < 1.0, 0.5 * d * d, ad - 0.5)
    row_sl1 = jnp.sum(sl1, axis=2)
    part = jnp.sum(jnp.where(lab > 0, row_sl1, 0.0))

    @pl.when(step == 0)
    def _():
        sl1_ref[...] = jnp.zeros((1, 1), jnp.float32)

    sl1_ref[...] += part.reshape(1, 1)


def _sum11(v):
    return jnp.sum(jnp.sum(v, axis=1, keepdims=True), axis=0, keepdims=True)


def _select_pass_kernel(mining_ref, ce_ref, lab_ref, sl1_ref,
                        reg_ref, cls_ref, tot_ref, acc_cls, acc_pos, *,
                        batch, neg_pos_ratio):
    i = pl.program_id(0)
    mining = mining_ref[0]
    ce = ce_ref[0]
    lab = lab_ref[0]
    rows, l = mining.shape
    p = rows * l

    pos = lab > 0
    num_pos = _sum11(pos.astype(jnp.int32))
    quota = num_pos * neg_pos_ratio
    neg_count = p - num_pos
    take_all = quota >= neg_count

    ib = jax.lax.bitcast_convert_type(mining, jnp.int32)
    key = jnp.where(pos, jnp.full_like(ib, _INT_MIN), ib)

    def vbody(j, t):
        cand = t | jax.lax.shift_left(jnp.int32(1), jnp.int32(30) - j)
        cnt = _sum11((key >= cand).astype(jnp.int32))
        return jnp.where(cnt >= quota, cand, t)

    t = jax.lax.fori_loop(0, 31, vbody, jnp.zeros((1, 1), jnp.int32))
    count_gt = _sum11((key > t).astype(jnp.int32))
    r = quota - count_gt
    tie = key == t
    idx = (jax.lax.broadcasted_iota(jnp.int32, (rows, l), 0) * l
           + jax.lax.broadcasted_iota(jnp.int32, (rows, l), 1))

    def ibody(j, s):
        cand = s | jax.lax.shift_left(jnp.int32(1), jnp.int32(13) - j)
        cnt = _sum11((tie & (idx < cand)).astype(jnp.int32))
        return jnp.where(cnt < r, cand, s)

    s = jax.lax.fori_loop(0, 14, ibody, jnp.zeros((1, 1), jnp.int32))

    inc = pos | take_all | (key > t) | (tie & (idx <= s) & (r > 0))
    part_cls = _sum11(jnp.where(inc, ce, 0.0))

    @pl.when(i == 0)
    def _():
        acc_cls[...] = jnp.zeros((1, 1), jnp.float32)
        acc_pos[...] = jnp.zeros((1, 1), jnp.float32)

    acc_cls[...] += part_cls
    acc_pos[...] += num_pos.astype(jnp.float32)

    @pl.when(i == batch - 1)
    def _():
        denom = jnp.maximum(acc_pos[...], 1.0)
        reg = sl1_ref[...] / denom
        cls = acc_cls[...] / denom
        reg_ref[...] = reg
        cls_ref[...] = cls
        tot_ref[...] = reg + cls


def kernel(confidence, predicted_locations, gt_locations, labels):
    b, p, c = confidence.shape
    n = b * p
    q = n // 128
    neg_pos_ratio = 3

    conf3 = confidence.reshape(q, 128, c)
    pred3 = predicted_locations.reshape(q, 128, 4)
    gt3 = gt_locations.reshape(q, 128, 4)
    lab_i32 = labels.astype(jnp.int32)
    lab2 = lab_i32.reshape(q, 128)

    bq = 64
    grid = (q // bq,)
    mining, ce, sl1 = pl.pallas_call(
        _dense_pass_kernel,
        grid=grid,
        in_specs=[
            pl.BlockSpec((bq, 128, c), lambda i: (i, 0, 0)),
            pl.BlockSpec((bq, 128), lambda i: (i, 0)),
            pl.BlockSpec((bq, 128, 4), lambda i: (i, 0, 0)),
            pl.BlockSpec((bq, 128, 4), lambda i: (i, 0, 0)),
        ],
        out_specs=[
            pl.BlockSpec((bq, 128), lambda i: (i, 0)),
            pl.BlockSpec((bq, 128), lambda i: (i, 0)),
            pl.BlockSpec((1, 1), lambda i: (0, 0)),
        ],
        out_shape=[
            jax.ShapeDtypeStruct((q, 128), jnp.float32),
            jax.ShapeDtypeStruct((q, 128), jnp.float32),
            jax.ShapeDtypeStruct((1, 1), jnp.float32),
        ],
    )(conf3, lab2, pred3, gt3)

    rows = p // 128
    mining3 = mining.reshape(b, rows, 128)
    ce3 = ce.reshape(b, rows, 128)
    lab3 = lab_i32.reshape(b, rows, 128)

    reg, cls, tot = pl.pallas_call(
        functools.partial(_select_pass_kernel, batch=b,
                          neg_pos_ratio=neg_pos_ratio),
        grid=(b,),
        in_specs=[
            pl.BlockSpec((1, rows, 128), lambda i: (i, 0, 0)),
            pl.BlockSpec((1, rows, 128), lambda i: (i, 0, 0)),
            pl.BlockSpec((1, rows, 128), lambda i: (i, 0, 0)),
            pl.BlockSpec((1, 1), lambda i: (0, 0)),
        ],
        out_specs=[
            pl.BlockSpec((1, 1), lambda i: (0, 0)),
            pl.BlockSpec((1, 1), lambda i: (0, 0)),
            pl.BlockSpec((1, 1), lambda i: (0, 0)),
        ],
        out_shape=[
            jax.ShapeDtypeStruct((1, 1), jnp.float32),
            jax.ShapeDtypeStruct((1, 1), jnp.float32),
            jax.ShapeDtypeStruct((1, 1), jnp.float32),
        ],
        scratch_shapes=[
            pltpu.VMEM((1, 1), jnp.float32),
            pltpu.VMEM((1, 1), jnp.float32),
        ],
    )(mining3, ce3, lab3, sl1)

    return (reg[0, 0], cls[0, 0], tot[0, 0])
